# baseline (speedup 1.0000x reference)
import functools

import numpy as np
import jax
import jax.numpy as jnp
from jax import lax
from jax.experimental import pallas as pl
from jax.experimental.pallas import tpu as pltpu
from jax.experimental.pallas import tpu_sc as plsc

N = 64
NSCALE = 4

def _toff(r, p):
    return 256 + (r * 6 + (p - 1)) * 80

W_SIZE = 256 + NSCALE * 6 * 80


def _build_static():
    tab = []
    for r in range(NSCALE):
        for p in range(1, (6 - r) + 1):
            length = 65 - (1 << p)
            prev = _toff(r, p - 1) if p > 1 else r * 64
            cur = _toff(r, p)
            h = 1 << (p - 1)
            for s0 in range(0, length, 16):
                tab.append((cur + s0, prev + s0, prev + s0 + h))
    vops = []
    for r in range(NSCALE):
        beta = 1 << r
        S = N >> r
        for k in range(S):
            s = k * beta
            for L in range(1, S - k + 1):
                e = s + L * beta
                if L == 1:
                    a = b = r * 64 + s
                else:
                    p = L.bit_length() - 1
                    a = _toff(r, p) + s
                    b = _toff(r, p) + s + L - (1 << p)
                vops.append((r, s, e, a, b))
    while len(vops) % 16:
        c0 = (len(vops) // 16) * 16
        vops.append(vops[c0])
    return tab, vops


_TAB, _VOPS = _build_static()
_VN = len(_VOPS) // 16
_IDX_ROWS = (_VN * 16 + 127) // 128 + 1


def _pad_rows(vals):
    out = np.zeros((_IDX_ROWS * 128,), np.int32)
    out[:len(vals)] = vals
    return out.reshape(_IDX_ROWS, 128)


_VR = _pad_rows([o[0] for o in _VOPS])
_VS = _pad_rows([o[1] for o in _VOPS])
_VE = _pad_rows([o[2] for o in _VOPS])
_VA = _pad_rows([o[3] for o in _VOPS])
_VB = _pad_rows([o[4] for o in _VOPS])

_NC, _NS = 2, 16
_NW = _NC * _NS
_B, _D = 4, 256
_PAIRS = _B * _D
_PER_W = _PAIRS // _NW


def _idx_vec(ref, v):
    return ref[v // 8, pl.ds((v % 8) * 16, 16)]


def _sc_body(xr, vr, vs, ve, va, vb, hid,
             w, stage, plane0, plane1,
             vrv, vsv, vev, vav, vbv,
             sem_h0, sem_h1):
    wid = lax.axis_index("s") * _NC + lax.axis_index("c")
    base = wid * _PER_W
    pltpu.sync_copy(vr, vrv)
    pltpu.sync_copy(vs, vsv)
    pltpu.sync_copy(ve, vev)
    pltpu.sync_copy(va, vav)
    pltpu.sync_copy(vb, vbv)

    zero = jnp.zeros((16,), jnp.float32)

    @pl.loop(0, N)
    def _(s):
        for r in range(NSCALE):
            for c in (0, 16, 32, 48, 49):
                plane0[r, s, pl.ds(c, 16)] = zero
                plane1[r, s, pl.ds(c, 16)] = zero

    planes = (plane0, plane1)
    sems = (sem_h0, sem_h1)

    @pl.loop(0, _PER_W // 2)
    def _(g):
        for b2 in range(2):
            i = g * 2 + b2
            p = base + i
            bi = p // _D
            di = p % _D
            plane = planes[b2]
            sem = sems[b2]

            @pl.when(g > 0)
            def _():
                pltpu.make_async_copy(plane, hid.at[bi, di], sem).wait()

            pltpu.sync_copy(xr.at[pl.ds(2 * p, 2)], stage)
            for c in range(16):
                w[pl.ds(c * 16, 16)] = stage[c // 8, pl.ds((c % 8) * 16, 16)]
            for dst, a, b in _TAB:
                w[pl.ds(dst, 16)] = jnp.maximum(w[pl.ds(a, 16)],
                                                w[pl.ds(b, 16)])
            for v in range(_VN):
                vals = jnp.maximum(
                    plsc.load_gather(w, [_idx_vec(vav, v)]),
                    plsc.load_gather(w, [_idx_vec(vbv, v)]))
                plsc.store_scatter(
                    plane,
                    [_idx_vec(vrv, v), _idx_vec(vsv, v), _idx_vec(vev, v)],
                    vals)

            pltpu.async_copy(plane, hid.at[bi, di], sem)

    bi0 = base // _D
    di0 = base % _D
    for b2 in range(2):
        pltpu.make_async_copy(planes[b2], hid.at[bi0, di0], sems[b2]).wait()


def _tc_mask_body(msk):
    s_i = lax.broadcasted_iota(jnp.int32, (N, N + 1), 0)
    e_i = lax.broadcasted_iota(jnp.int32, (N, N + 1), 1)
    for r in range(NSCALE):
        beta = 1 << r
        m = ((s_i & (beta - 1)) == 0) & (e_i > s_i) \
            & (((e_i - s_i) & (beta - 1)) == 0)
        msk[0, 0, r] = m.astype(jnp.float32)


@jax.jit
def _run(xr, vr, vs, ve, va, vb):
    f = pl.kernel(
        _sc_body,
        out_type=jax.ShapeDtypeStruct((_B, _D, NSCALE, N, N + 1),
                                      jnp.float32),
        mesh=plsc.VectorSubcoreMesh(core_axis_name="c", subcore_axis_name="s"),
        compiler_params=pltpu.CompilerParams(needs_layout_passes=False,
                                             use_tc_tiling_on_sc=True),
        scratch_types=[
            pltpu.VMEM((W_SIZE,), jnp.float32),
            pltpu.VMEM((2, 128), jnp.float32),
            pltpu.VMEM((NSCALE, N, N + 1), jnp.float32),
            pltpu.VMEM((NSCALE, N, N + 1), jnp.float32),
            pltpu.VMEM((_IDX_ROWS, 128), jnp.int32),
            pltpu.VMEM((_IDX_ROWS, 128), jnp.int32),
            pltpu.VMEM((_IDX_ROWS, 128), jnp.int32),
            pltpu.VMEM((_IDX_ROWS, 128), jnp.int32),
            pltpu.VMEM((_IDX_ROWS, 128), jnp.int32),
            pltpu.SemaphoreType.DMA,
            pltpu.SemaphoreType.DMA,
        ],
    )
    hid = f(xr, vr, vs, ve, va, vb)
    msk = pl.pallas_call(
        _tc_mask_body,
        grid=(_B, _D),
        out_shape=jax.ShapeDtypeStruct((_B, _D, NSCALE, N, N + 1),
                                       jnp.float32),
        out_specs=pl.BlockSpec((1, 1, NSCALE, N, N + 1),
                               lambda b, d: (b, d, 0, 0, 0)),
    )()
    return hid, msk


def kernel(x):
    B, ns, D, n = x.shape
    xr = x[:, :NSCALE].transpose(0, 2, 1, 3).reshape(B * D * NSCALE * n // 128,
                                                     128)
    return _run(xr, jnp.asarray(_VR), jnp.asarray(_VS), jnp.asarray(_VE),
                jnp.asarray(_VA), jnp.asarray(_VB))


# speedup vs baseline: 1.7175x; 1.7175x over previous
import functools

import numpy as np
import jax
import jax.numpy as jnp
from jax import lax
from jax.experimental import pallas as pl
from jax.experimental.pallas import tpu as pltpu
from jax.experimental.pallas import tpu_sc as plsc

N = 64
NSCALE = 4
_B, _D = 4, 256
_NC = 2


def _sc_hidden_body(xt, out, xbuf, plane):
    wid = lax.axis_index("s") * _NC + lax.axis_index("c")
    b = wid // 8
    rem = wid % 8
    r = rem // 2
    dh = rem % 2
    beta = 1 << r

    pltpu.sync_copy(xt.at[b, r, dh], xbuf)

    zero = jnp.zeros((16,), jnp.float32)

    @pl.loop(0, N + 1)
    def _(e):
        for s_loc in range(8):
            for j in range(8):
                plane[e, s_loc, pl.ds(j * 16, 16)] = zero

    @pl.loop(0, N // 8)
    def _(i):
        @pl.when(i > 0)
        def _():
            for s_loc in range(8):
                for j in range(1, 9):
                    e = (i - 1) * 8 + s_loc + j
                    for q in range(8):
                        plane[e, s_loc, pl.ds(q * 16, 16)] = zero

        for s_loc in range(8):
            s = i * 8 + s_loc

            @pl.when(lax.rem(s, beta) == 0)
            def _():
                lmax = (N - s) // beta
                m0 = tuple(xbuf[s, pl.ds(j * 16, 16)] for j in range(8))

                def inner(l, m):
                    row = s + l - 1
                    e = s + l * beta
                    new = tuple(
                        jnp.maximum(m[j], xbuf[row, pl.ds(j * 16, 16)])
                        for j in range(8))
                    for j in range(8):
                        plane[e, s_loc, pl.ds(j * 16, 16)] = new[j]
                    return new

                pl.loop(1, lmax + 1, init_carry=m0)(inner)

        pltpu.sync_copy(
            plane,
            out.at[b, r, slice(None), pl.ds(i * 8, 8), pl.ds(dh * 128, 128)])


def _tc_mask_body(msk):
    r = pl.program_id(1)
    e = pl.program_id(2)
    beta = jnp.int32(1) << r
    s_i = lax.broadcasted_iota(jnp.int32, (N, _D), 0)
    m = ((s_i & (beta - 1)) == 0) & (e > s_i) \
        & (((e - s_i) & (beta - 1)) == 0)
    msk[0, 0, 0] = m.astype(jnp.float32)


@jax.jit
def _run(xt):
    hid_p = pl.kernel(
        _sc_hidden_body,
        out_type=jax.ShapeDtypeStruct((_B, NSCALE, N + 1, N, _D),
                                      jnp.float32),
        mesh=plsc.VectorSubcoreMesh(core_axis_name="c", subcore_axis_name="s"),
        compiler_params=pltpu.CompilerParams(needs_layout_passes=False,
                                             use_tc_tiling_on_sc=True),
        scratch_types=[
            pltpu.VMEM((N, 128), jnp.float32),
            pltpu.VMEM((N + 1, 8, 128), jnp.float32),
        ],
    )(xt)
    msk_p = pl.pallas_call(
        _tc_mask_body,
        grid=(_B, NSCALE, N + 1),
        out_shape=jax.ShapeDtypeStruct((_B, NSCALE, N + 1, N, _D),
                                       jnp.float32),
        out_specs=pl.BlockSpec((1, 1, 1, N, _D),
                               lambda b, r, e: (b, r, e, 0, 0)),
    )()
    perm = (0, 4, 1, 3, 2)
    return jnp.transpose(hid_p, perm), jnp.transpose(msk_p, perm)


def kernel(x):
    xt = x[:, :NSCALE].reshape(_B, NSCALE, 2, 128, N).transpose(0, 1, 2, 4, 3)
    return _run(xt)


# speedup vs baseline: 8.2957x; 4.8300x over previous
import functools

import numpy as np
import jax
import jax.numpy as jnp
from jax import lax
from jax.experimental import pallas as pl
from jax.experimental.pallas import tpu as pltpu
from jax.experimental.pallas import tpu_sc as plsc

N = 64
NSCALE = 4
_B, _D = 4, 256
_NC = 2


def _sc_hidden_body(xt, out, xbuf, plane):
    wid = lax.axis_index("s") * _NC + lax.axis_index("c")
    b = wid // 8
    rem = wid % 8
    r = rem // 2
    dh = rem % 2
    beta = 1 << r

    pltpu.sync_copy(xt.at[b, r, dh], xbuf)

    zero = jnp.zeros((16,), jnp.float32)

    @pl.loop(0, N + 1)
    def _(e):
        for s_loc in range(8):
            for j in range(8):
                plane[e, s_loc, pl.ds(j * 16, 16)] = zero

    @pl.loop(0, N // 8)
    def _(i):
        @pl.when(i > 0)
        def _():
            for s_loc in range(8):
                for j in range(1, 9):
                    e = (i - 1) * 8 + s_loc + j
                    for q in range(8):
                        plane[e, s_loc, pl.ds(q * 16, 16)] = zero

        for s_loc in range(8):
            s = i * 8 + s_loc

            @pl.when(lax.rem(s, beta) == 0)
            def _():
                lmax = (N - s) // beta
                m0 = tuple(xbuf[s, pl.ds(j * 16, 16)] for j in range(8))

                def inner(l, m):
                    row = s + l - 1
                    e = s + l * beta
                    new = tuple(
                        jnp.maximum(m[j], xbuf[row, pl.ds(j * 16, 16)])
                        for j in range(8))
                    for j in range(8):
                        plane[e, s_loc, pl.ds(j * 16, 16)] = new[j]
                    return new

                pl.loop(1, lmax + 1, init_carry=m0)(inner)

        pltpu.sync_copy(
            plane,
            out.at[b, r, slice(None), pl.ds(i * 8, 8), pl.ds(dh * 128, 128)])


_E_BLK = 5


def _tc_mask_body(msk):
    ec = pl.program_id(0)
    s_i = lax.broadcasted_iota(jnp.int32, (N, _D), 0)
    for r in range(NSCALE):
        beta = 1 << r
        for de in range(_E_BLK):
            e = ec * _E_BLK + de
            m = (((s_i & (beta - 1)) == 0) & (e > s_i)
                 & (((e - s_i) & (beta - 1)) == 0)).astype(jnp.float32)
            for b in range(_B):
                msk[b, r, de] = m


@jax.jit
def _run(xt):
    hid_p = pl.kernel(
        _sc_hidden_body,
        out_type=jax.ShapeDtypeStruct((_B, NSCALE, N + 1, N, _D),
                                      jnp.float32),
        mesh=plsc.VectorSubcoreMesh(core_axis_name="c", subcore_axis_name="s"),
        compiler_params=pltpu.CompilerParams(needs_layout_passes=False,
                                             use_tc_tiling_on_sc=True),
        scratch_types=[
            pltpu.VMEM((N, 128), jnp.float32),
            pltpu.VMEM((N + 1, 8, 128), jnp.float32),
        ],
    )(xt)
    msk_p = pl.pallas_call(
        _tc_mask_body,
        grid=((N + 1) // _E_BLK,),
        out_shape=jax.ShapeDtypeStruct((_B, NSCALE, N + 1, N, _D),
                                       jnp.float32),
        out_specs=pl.BlockSpec((_B, NSCALE, _E_BLK, N, _D),
                               lambda ec: (0, 0, ec, 0, 0)),
    )()
    perm = (0, 4, 1, 3, 2)
    return jnp.transpose(hid_p, perm), jnp.transpose(msk_p, perm)


def kernel(x):
    xt = x[:, :NSCALE].reshape(_B, NSCALE, 2, 128, N).transpose(0, 1, 2, 4, 3)
    return _run(xt)
